# baseline (speedup 1.0000x reference)
import jax
import jax.numpy as jnp
from jax import lax
from jax.experimental import pallas as pl

NUM_REGIONS = 6
EMBED_DIM = 512
BATCH = 4096
ROW = NUM_REGIONS * EMBED_DIM
BLK = 512


def _body(ids_ref, w_ref, out_ref):
    ids = ids_ref[...]
    iota = lax.broadcasted_iota(jnp.int32, (NUM_REGIONS, NUM_REGIONS), 1)
    oh = (ids == iota).astype(jnp.float32)
    tokens = jnp.dot(oh, w_ref[...], preferred_element_type=jnp.float32)
    for r in range(NUM_REGIONS):
        row = lax.slice(tokens, (r, 0), (r + 1, EMBED_DIM))
        out_ref[:, r * EMBED_DIM:(r + 1) * EMBED_DIM] = jnp.broadcast_to(
            row, (BLK, EMBED_DIM))


def kernel(token_embed_weight, region_ids, batch_size):
    del batch_size
    ids2 = region_ids.astype(jnp.int32).reshape(NUM_REGIONS, 1)
    out2 = pl.pallas_call(
        _body,
        grid=(BATCH // BLK,),
        in_specs=[
            pl.BlockSpec((NUM_REGIONS, 1), lambda i: (0, 0)),
            pl.BlockSpec((NUM_REGIONS, EMBED_DIM), lambda i: (0, 0)),
        ],
        out_specs=pl.BlockSpec((BLK, ROW), lambda i: (i, 0)),
        out_shape=jax.ShapeDtypeStruct((BATCH, ROW), jnp.float32),
    )(ids2, token_embed_weight)
    return out2.reshape(BATCH, NUM_REGIONS, EMBED_DIM)


# speedup vs baseline: 1.8355x; 1.8355x over previous
import jax
import jax.numpy as jnp
from jax import lax
from jax.experimental import pallas as pl

NUM_REGIONS = 6
EMBED_DIM = 512
BATCH = 4096
ROW = NUM_REGIONS * EMBED_DIM
BLK = 512


def _body(ids_ref, w_ref, out_ref):
    ids = ids_ref[...]
    iota = lax.broadcasted_iota(jnp.int32, (NUM_REGIONS, NUM_REGIONS), 1)
    oh = (ids == iota).astype(jnp.float32)
    tokens = jnp.dot(oh, w_ref[...], preferred_element_type=jnp.float32)
    out_ref[...] = jnp.broadcast_to(tokens[None], (BLK, NUM_REGIONS, EMBED_DIM))


def kernel(token_embed_weight, region_ids, batch_size):
    del batch_size
    ids2 = region_ids.astype(jnp.int32).reshape(NUM_REGIONS, 1)
    return pl.pallas_call(
        _body,
        grid=(BATCH // BLK,),
        in_specs=[
            pl.BlockSpec((NUM_REGIONS, 1), lambda i: (0, 0)),
            pl.BlockSpec((NUM_REGIONS, EMBED_DIM), lambda i: (0, 0)),
        ],
        out_specs=pl.BlockSpec((BLK, NUM_REGIONS, EMBED_DIM),
                               lambda i: (i, 0, 0)),
        out_shape=jax.ShapeDtypeStruct((BATCH, NUM_REGIONS, EMBED_DIM),
                                       jnp.float32),
    )(ids2, token_embed_weight)


# speedup vs baseline: 1.8388x; 1.0018x over previous
import jax
import jax.numpy as jnp
from jax import lax
from jax.experimental import pallas as pl
from jax.experimental.pallas import tpu as pltpu

NUM_REGIONS = 6
EMBED_DIM = 512
BATCH = 4096
R = 128
NCOPY = BATCH // R


def _body(ids_ref, w_ref, out_ref, buf_ref, sems):
    ids = ids_ref[...]
    iota = lax.broadcasted_iota(jnp.int32, (NUM_REGIONS, NUM_REGIONS), 1)
    oh = (ids == iota).astype(jnp.float32)
    tokens = jnp.dot(oh, w_ref[...], preferred_element_type=jnp.float32,
                     precision=lax.Precision.HIGHEST)
    buf_ref[...] = jnp.broadcast_to(tokens[None],
                                    (R, NUM_REGIONS, EMBED_DIM))
    for j in range(NCOPY):
        pltpu.make_async_copy(
            buf_ref, out_ref.at[pl.ds(j * R, R)], sems.at[j]).start()
    for j in range(NCOPY):
        pltpu.make_async_copy(
            buf_ref, out_ref.at[pl.ds(j * R, R)], sems.at[j]).wait()


def kernel(token_embed_weight, region_ids, batch_size):
    del batch_size
    ids2 = region_ids.astype(jnp.int32).reshape(NUM_REGIONS, 1)
    return pl.pallas_call(
        _body,
        in_specs=[
            pl.BlockSpec(memory_space=pltpu.VMEM),
            pl.BlockSpec(memory_space=pltpu.VMEM),
        ],
        out_specs=pl.BlockSpec(memory_space=pl.ANY),
        out_shape=jax.ShapeDtypeStruct((BATCH, NUM_REGIONS, EMBED_DIM),
                                       jnp.float32),
        scratch_shapes=[
            pltpu.VMEM((R, NUM_REGIONS, EMBED_DIM), jnp.float32),
            pltpu.SemaphoreType.DMA((NCOPY,)),
        ],
    )(ids2, token_embed_weight)
